# baseline (speedup 1.0000x reference)
import functools

import jax
import jax.numpy as jnp
from jax import lax
from jax.experimental import pallas as pl
from jax.experimental.pallas import tpu as pltpu
from jax.experimental.pallas import tpu_sc as plsc

_CHUNK = 128
_NBUF = 3
_NL = 16


def _build_sc_kernel(n_rows, emb, n_posseg):
  info = plsc.get_sparse_core_info()
  nc, ns = info.num_cores, info.num_subcores
  nw = nc * ns
  chunk = _CHUNK
  assert n_rows % (nw * chunk) == 0
  rows_per_w = n_rows // nw
  n_chunks = rows_per_w // chunk
  slab = chunk * emb

  mesh = plsc.VectorSubcoreMesh(core_axis_name="c", subcore_axis_name="s")

  @functools.partial(
      pl.kernel,
      mesh=mesh,
      out_type=jax.ShapeDtypeStruct((n_rows * emb,), jnp.float32),
      scratch_types=[
          pltpu.VMEM((_NBUF, chunk), jnp.int32),
          pltpu.VMEM((_NBUF, chunk), jnp.int32),
          pltpu.VMEM((_NBUF, chunk, 2 * emb), jnp.float32),
          pltpu.VMEM((n_posseg,), jnp.float32),
          pltpu.VMEM((2 * slab,), jnp.float32),
          pltpu.SemaphoreType.DMA((_NBUF,)),
          pltpu.SemaphoreType.DMA((_NBUF,)),
          pltpu.SemaphoreType.DMA((2,)),
          pltpu.SemaphoreType.DMA,
      ],
  )
  def gather_sum(idx_hbm, aux_hbm, table_hbm, posseg_hbm, out_hbm,
                 idx_v, aux_v, tok_v, ps_v, out_v,
                 sem_i, sem_g, sem_o, sem_p):
    wid = lax.axis_index("s") * nc + lax.axis_index("c")
    base_c = wid * n_chunks

    pltpu.async_copy(posseg_hbm, ps_v, sem_p)

    def issue_idx(c, b):
      e0 = (base_c + c) * chunk
      pltpu.async_copy(idx_hbm.at[pl.ds(e0, chunk)], idx_v.at[b], sem_i.at[b])
      pltpu.async_copy(aux_hbm.at[pl.ds(e0, chunk)], aux_v.at[b], sem_i.at[b])

    def wait_idx(c, b):
      e0 = (base_c + c) * chunk
      pltpu.make_async_copy(
          idx_hbm.at[pl.ds(e0, chunk)], idx_v.at[b], sem_i.at[b]).wait()
      pltpu.make_async_copy(
          aux_hbm.at[pl.ds(e0, chunk)], aux_v.at[b], sem_i.at[b]).wait()

    def issue_gather(b):
      pltpu.async_copy(
          table_hbm.at[idx_v.at[b]], tok_v.at[b], sem_g.at[b])

    def wait_gather(b):
      pltpu.make_async_copy(
          table_hbm.at[idx_v.at[b]], tok_v.at[b], sem_g.at[b]).wait()

    def issue_store(c, s):
      pltpu.async_copy(
          out_v.at[pl.ds(s * slab, slab)],
          out_hbm.at[pl.ds((base_c + c) * slab, slab)], sem_o.at[s])

    def wait_store(c, s):
      pltpu.make_async_copy(
          out_v.at[pl.ds(s * slab, slab)],
          out_hbm.at[pl.ds((base_c + c) * slab, slab)], sem_o.at[s]).wait()

    issue_idx(0, 0)
    issue_idx(1, 1)
    wait_idx(0, 0)
    issue_gather(0)
    pltpu.make_async_copy(posseg_hbm, ps_v, sem_p).wait()

    def chunk_body(c, _):
      b = lax.rem(c, _NBUF)
      s = lax.rem(c, 2)

      @pl.when(c + 2 < n_chunks)
      def _():
        issue_idx(c + 2, lax.rem(c + 2, _NBUF))

      @pl.when(c + 1 < n_chunks)
      def _():
        b1 = lax.rem(c + 1, _NBUF)
        wait_idx(c + 1, b1)
        issue_gather(b1)

      wait_gather(b)

      @pl.when(c >= 2)
      def _():
        wait_store(c - 2, s)

      tb = tok_v.at[b]
      obase = s * slab

      def grp_body(q, _):
        r0 = q * _NL
        auxv = aux_v[b, pl.ds(r0, _NL)]
        for rr in range(_NL):
          so_p = auxv[rr]
          h = lax.rem(so_p, 2)
          so = so_p - h
          r = r0 + rr
          for g in range(emb // _NL):
            v0 = tb[r, pl.ds(g * _NL, _NL)]
            v1 = tb[r, pl.ds(emb + g * _NL, _NL)]
            v = (jnp.where(h != 0, v1, v0)
                 + ps_v[pl.ds(so + g * _NL, _NL)])
            out_v[pl.ds(obase + r * emb + g * _NL, _NL)] = v
        return ()

      lax.fori_loop(0, chunk // _NL, grp_body, ())
      issue_store(c, s)
      return ()

    lax.fori_loop(0, n_chunks, chunk_body, ())
    wait_store(n_chunks - 2, 0)
    wait_store(n_chunks - 1, 1)

  return gather_sum


def kernel(sequence, segment_label, token_weight, pos_weight, seg_weight):
  bsz, seq_len = sequence.shape
  n_vocab, emb = token_weight.shape
  n_seg = seg_weight.shape[0]
  n_rows = bsz * seq_len

  table2 = token_weight.reshape(n_vocab // 2, 2 * emb)
  seq32 = sequence.astype(jnp.int32)
  idx = (seq32 >> 1).reshape(-1)
  aux = ((n_seg * jnp.arange(seq_len, dtype=jnp.int32)[None, :]
          + segment_label.astype(jnp.int32)) * emb
         + (seq32 & 1)).reshape(-1)
  posseg = (pos_weight[:seq_len, None, :] + seg_weight[None, :, :]).reshape(-1)

  sc = _build_sc_kernel(n_rows, emb, seq_len * n_seg * emb)
  out = sc(idx, aux, table2, posseg)
  return out.reshape(bsz, seq_len, emb)


# speedup vs baseline: 1.0807x; 1.0807x over previous
import functools

import jax
import jax.numpy as jnp
from jax import lax
from jax.experimental import pallas as pl
from jax.experimental.pallas import tpu as pltpu
from jax.experimental.pallas import tpu_sc as plsc

_CHUNK = 128
_NBUF = 4
_NL = 16


def _build_sc_kernel(n_rows, emb, n_posseg):
  info = plsc.get_sparse_core_info()
  nc, ns = info.num_cores, info.num_subcores
  nw = nc * ns
  chunk = _CHUNK
  assert n_rows % (nw * chunk) == 0
  rows_per_w = n_rows // nw
  n_chunks = rows_per_w // chunk
  slab = chunk * emb

  mesh = plsc.VectorSubcoreMesh(core_axis_name="c", subcore_axis_name="s")

  @functools.partial(
      pl.kernel,
      mesh=mesh,
      compiler_params=pltpu.CompilerParams(use_tc_tiling_on_sc=True),
      out_type=jax.ShapeDtypeStruct((n_rows * emb,), jnp.float32),
      scratch_types=[
          pltpu.VMEM((_NBUF, chunk), jnp.int32),
          pltpu.VMEM((_NBUF, chunk), jnp.int32),
          pltpu.VMEM((_NBUF, chunk, 2 * emb), jnp.float32),
          pltpu.VMEM((n_posseg,), jnp.float32),
          pltpu.VMEM((2 * slab,), jnp.float32),
          pltpu.SemaphoreType.DMA((_NBUF,)),
          pltpu.SemaphoreType.DMA((_NBUF,)),
          pltpu.SemaphoreType.DMA((2,)),
          pltpu.SemaphoreType.DMA,
      ],
  )
  def gather_sum(idx_hbm, aux_hbm, table_hbm, posseg_hbm, out_hbm,
                 idx_v, aux_v, tok_v, ps_v, out_v,
                 sem_i, sem_g, sem_o, sem_p):
    wid = lax.axis_index("s") * nc + lax.axis_index("c")
    base_c = wid * n_chunks

    pltpu.async_copy(posseg_hbm, ps_v, sem_p)

    def issue_idx(c, b):
      ci = base_c + c
      pltpu.async_copy(idx_hbm.at[ci], idx_v.at[b], sem_i.at[b])
      pltpu.async_copy(aux_hbm.at[ci], aux_v.at[b], sem_i.at[b])

    def wait_idx(c, b):
      ci = base_c + c
      pltpu.make_async_copy(
          idx_hbm.at[ci], idx_v.at[b], sem_i.at[b]).wait()
      pltpu.make_async_copy(
          aux_hbm.at[ci], aux_v.at[b], sem_i.at[b]).wait()

    def issue_gather(b):
      pltpu.async_copy(
          table_hbm.at[idx_v.at[b]], tok_v.at[b], sem_g.at[b])

    def wait_gather(b):
      pltpu.make_async_copy(
          table_hbm.at[idx_v.at[b]], tok_v.at[b], sem_g.at[b]).wait()

    def issue_store(c, s):
      pltpu.async_copy(
          out_v.at[pl.ds(s * slab, slab)],
          out_hbm.at[pl.ds((base_c + c) * slab, slab)], sem_o.at[s])

    def wait_store(c, s):
      pltpu.make_async_copy(
          out_v.at[pl.ds(s * slab, slab)],
          out_hbm.at[pl.ds((base_c + c) * slab, slab)], sem_o.at[s]).wait()

    issue_idx(0, 0)
    issue_idx(1, 1)
    issue_idx(2, 2)
    wait_idx(0, 0)
    issue_gather(0)
    wait_idx(1, 1)
    issue_gather(1)
    pltpu.make_async_copy(posseg_hbm, ps_v, sem_p).wait()

    def chunk_body(c, _):
      b = lax.rem(c, _NBUF)
      s = lax.rem(c, 2)

      @pl.when(c + 3 < n_chunks)
      def _():
        issue_idx(c + 3, lax.rem(c + 3, _NBUF))

      @pl.when(c + 2 < n_chunks)
      def _():
        b2 = lax.rem(c + 2, _NBUF)
        wait_idx(c + 2, b2)
        issue_gather(b2)

      wait_gather(b)

      @pl.when(c >= 2)
      def _():
        wait_store(c - 2, s)

      tb = tok_v.at[b]
      obase = s * slab

      def grp_body(q, _):
        r0 = q * _NL
        auxv = aux_v[b, pl.ds(r0, _NL)]
        for rr in range(_NL):
          so = auxv[rr]
          r = r0 + rr
          for g in range(emb // _NL):
            v = (tb[r, pl.ds(g * _NL, _NL)]
                 + ps_v[pl.ds(so + g * _NL, _NL)])
            out_v[pl.ds(obase + r * emb + g * _NL, _NL)] = v
        return ()

      lax.fori_loop(0, chunk // _NL, grp_body, ())
      issue_store(c, s)
      return ()

    lax.fori_loop(0, n_chunks, chunk_body, ())
    wait_store(n_chunks - 2, 0)
    wait_store(n_chunks - 1, 1)

  return gather_sum


def kernel(sequence, segment_label, token_weight, pos_weight, seg_weight):
  bsz, seq_len = sequence.shape
  n_vocab, emb = token_weight.shape
  n_seg = seg_weight.shape[0]
  n_rows = bsz * seq_len
  n_chunks_total = n_rows // _CHUNK

  table2 = jnp.pad(token_weight, ((0, 0), (0, 2 * emb - emb)))
  posseg = (pos_weight[:seq_len, None, :] + seg_weight[None, :, :]).reshape(-1)
  idx = sequence.astype(jnp.int32).reshape(n_chunks_total, _CHUNK)
  aux = ((n_seg * jnp.arange(seq_len, dtype=jnp.int32)[None, :]
          + segment_label.astype(jnp.int32)) * emb
         ).reshape(n_chunks_total, _CHUNK)

  sc = _build_sc_kernel(n_rows, emb, seq_len * n_seg * emb)
  out = sc(idx, aux, table2, posseg)
  return out.reshape(bsz, seq_len, emb)
